# baseline (speedup 1.0000x reference)
import functools

import jax
import jax.numpy as jnp
import numpy as np
from jax import lax
from jax.experimental import pallas as pl
from jax.experimental.pallas import tpu as pltpu
from jax.experimental.pallas import tpu_sc as plsc

N = 10000
E = 320000
HID = 128
T = 13
STEPS = 6

BLK = 1000
NB = N // BLK
CHUNK = 128
NCH = E // CHUNK
NCORES = 2
NSUB = 16
NWORK = NCORES * NSUB
BASE_CH = NCH // NWORK
REM_CH = NCH % NWORK
SLOT = 80
NPAD = 10240
ROWS_PER_SUB = NPAD // NSUB
ZROWS = 128
ZCOPY = ROWS_PER_SUB // ZROWS

_STARTS = [w * BASE_CH + min(w, REM_CH) for w in range(NWORK)]
_COUNTS = [BASE_CH + (1 if w < REM_CH else 0) for w in range(NWORK)]
_ROW_MAP = np.concatenate(
    [np.minimum(np.arange(s, s + SLOT), NCH - 1) for s in _STARTS])
_PAD_ROW = np.concatenate(
    [np.arange(SLOT) >= c for c in _COUNTS])[:, None]
_PAD_DST = (N + np.arange(CHUNK))[None, :]


def _msg_body(h_ref, w_ref, b_ref, out_ref):
    hs = lax.dot_general(h_ref[...], w_ref[0], (((1,), (1,)), ((), ())),
                         preferred_element_type=jnp.float32)
    out_ref[0] = hs + b_ref[0]


def _msg_transform(h, W_msg, b_msg):
    return pl.pallas_call(
        _msg_body,
        grid=(NB, T),
        in_specs=[
            pl.BlockSpec((BLK, HID), lambda i, t: (i, 0)),
            pl.BlockSpec((1, HID, HID), lambda i, t: (t, 0, 0)),
            pl.BlockSpec((1, 1, HID), lambda i, t: (t, 0, 0)),
        ],
        out_specs=pl.BlockSpec((1, BLK, HID), lambda i, t: (t, i, 0)),
        out_shape=jax.ShapeDtypeStruct((T, N, HID), jnp.float32),
    )(h, W_msg, b_msg.reshape(T, 1, HID))


def _sc_scatter(hs_flat, gidx_p, dst_p):
    mesh = plsc.VectorSubcoreMesh(core_axis_name="c", subcore_axis_name="s")

    @functools.partial(
        pl.kernel,
        out_type=jax.ShapeDtypeStruct((NCORES, NPAD, HID), jnp.float32),
        mesh=mesh,
        scratch_types=[
            pltpu.VMEM((SLOT // 2, CHUNK), jnp.int32),
            pltpu.VMEM((SLOT // 2, CHUNK), jnp.int32),
            pltpu.VMEM((CHUNK, HID), jnp.float32),
            pltpu.VMEM_SHARED((NPAD, HID), jnp.float32),
        ],
    )
    def k(hs_hbm, gi_hbm, di_hbm, out_hbm, gi_v, di_v, buf, acc):
        c = lax.axis_index("c")
        s = lax.axis_index("s")
        wid = c * NSUB + s

        def zb(kk, _):
            buf[kk // 8, pl.ds((kk % 8) * 16, 16)] = jnp.zeros((16,), jnp.float32)
            return 0
        lax.fori_loop(0, (CHUNK * HID) // 16, zb, 0)
        base = s * ROWS_PER_SUB
        for kk in range(ZCOPY):
            pltpu.sync_copy(buf.at[pl.ds(0, ZROWS)],
                            acc.at[pl.ds(base + kk * ZROWS, ZROWS)])
        plsc.subcore_barrier()

        half = SLOT // 2

        def do_half(h0):
            pltpu.sync_copy(gi_hbm.at[pl.ds(wid * SLOT + h0, half)], gi_v)
            pltpu.sync_copy(di_hbm.at[pl.ds(wid * SLOT + h0, half)], di_v)

            def body(j, _):
                pltpu.sync_copy(hs_hbm.at[gi_v.at[j]], buf)
                pltpu.sync_copy(buf, acc.at[di_v.at[j]], add=True)
                return 0
            lax.fori_loop(0, half, body, 0)

        do_half(0)
        do_half(half)

        plsc.subcore_barrier()
        for kk in range(ZCOPY):
            r0 = base + kk * ZROWS
            pltpu.sync_copy(acc.at[pl.ds(r0, ZROWS)],
                            out_hbm.at[c, pl.ds(r0, ZROWS)])

    return k(hs_flat, gidx_p, dst_p)


def _gru_body(parts_ref, h_ref, wih_ref, whh_ref, bih_ref, bhh_ref, out_ref):
    a = parts_ref[0] + parts_ref[1]
    h = h_ref[...]

    def gates(x, w3, b2):
        return [lax.dot_general(x, w3[g], (((1,), (1,)), ((), ())),
                                preferred_element_type=jnp.float32)
                + b2[g][None, :] for g in range(3)]

    gi = gates(a, wih_ref, bih_ref)
    gh = gates(h, whh_ref, bhh_ref)
    r = jax.nn.sigmoid(gi[0] + gh[0])
    z = jax.nn.sigmoid(gi[1] + gh[1])
    cc = jnp.tanh(gi[2] + r * gh[2])
    out_ref[...] = (1.0 - z) * cc + z * h


def _gru(parts, h, wih3, whh3, bih2, bhh2):
    return pl.pallas_call(
        _gru_body,
        grid=(NB,),
        in_specs=[
            pl.BlockSpec((NCORES, BLK, HID), lambda i: (0, i, 0)),
            pl.BlockSpec((BLK, HID), lambda i: (i, 0)),
            pl.BlockSpec((3, HID, HID), lambda i: (0, 0, 0)),
            pl.BlockSpec((3, HID, HID), lambda i: (0, 0, 0)),
            pl.BlockSpec((3, HID), lambda i: (0, 0)),
            pl.BlockSpec((3, HID), lambda i: (0, 0)),
        ],
        out_specs=pl.BlockSpec((BLK, HID), lambda i: (i, 0)),
        out_shape=jax.ShapeDtypeStruct((N, HID), jnp.float32),
    )(parts, h, wih3, whh3, bih2, bhh2)


def _final_body(h_ref, wcls_ref, bcls_ref, out_ref, acc_ref):
    i = pl.program_id(0)

    @pl.when(i == 0)
    def _():
        acc_ref[...] = jnp.zeros_like(acc_ref)

    acc_ref[...] += jnp.sum(jax.nn.relu(h_ref[...]), axis=0, keepdims=True)

    @pl.when(i == NB - 1)
    def _():
        sv = jnp.sum(acc_ref[...] * wcls_ref[...]) + bcls_ref[0, 0]
        out_ref[...] = jax.nn.sigmoid(sv) * jnp.ones((1, 1), jnp.float32)


def _final(h, W_cls, b_cls):
    return pl.pallas_call(
        _final_body,
        grid=(NB,),
        in_specs=[
            pl.BlockSpec((BLK, HID), lambda i: (i, 0)),
            pl.BlockSpec((1, HID), lambda i: (0, 0)),
            pl.BlockSpec((1, 1), lambda i: (0, 0)),
        ],
        out_specs=pl.BlockSpec((1, 1), lambda i: (0, 0)),
        out_shape=jax.ShapeDtypeStruct((1, 1), jnp.float32),
        scratch_shapes=[pltpu.VMEM((1, HID), jnp.float32)],
    )(h, W_cls, b_cls.reshape(1, 1))


def kernel(x, edge_index, etypes, W_msg, b_msg, W_ih, W_hh, b_ih, b_hh, W_cls, b_cls):
    src = edge_index[0].astype(jnp.int32)
    dst = edge_index[1].astype(jnp.int32)
    et = etypes.astype(jnp.int32)

    row_map = jnp.asarray(_ROW_MAP, dtype=jnp.int32)
    pad_row = jnp.asarray(_PAD_ROW)
    pad_dst = jnp.asarray(_PAD_DST, dtype=jnp.int32)
    gidx_p = jnp.where(pad_row, 0, (et * N + src).reshape(NCH, CHUNK)[row_map])
    dst_p = jnp.where(pad_row, pad_dst, dst.reshape(NCH, CHUNK)[row_map])

    wih3 = W_ih.reshape(3, HID, HID)
    whh3 = W_hh.reshape(3, HID, HID)
    bih2 = b_ih.reshape(3, HID)
    bhh2 = b_hh.reshape(3, HID)

    h = x
    for _ in range(STEPS):
        hs = _msg_transform(h, W_msg, b_msg)
        parts = _sc_scatter(hs.reshape(T * N, HID), gidx_p, dst_p)
        h = _gru(parts, h, wih3, whh3, bih2, bhh2)

    return _final(h, W_cls, b_cls).reshape(1)


# speedup vs baseline: 2.0685x; 2.0685x over previous
import functools

import jax
import jax.numpy as jnp
import numpy as np
from jax import lax
from jax.experimental import pallas as pl
from jax.experimental.pallas import tpu as pltpu
from jax.experimental.pallas import tpu_sc as plsc

N = 10000
E = 320000
HID = 128
T = 13
STEPS = 6

BLK = 1000
NB = N // BLK
CHUNK = 128
NCH = E // CHUNK
NCORES = 2
NSUB = 16
NWORK = NCORES * NSUB
BASE_CH = NCH // NWORK
REM_CH = NCH % NWORK
SLOT = 80
NPAD = 10240
ROWS_PER_SUB = NPAD // NSUB
ZROWS = 128
ZCOPY = ROWS_PER_SUB // ZROWS

_STARTS = [w * BASE_CH + min(w, REM_CH) for w in range(NWORK)]
_COUNTS = [BASE_CH + (1 if w < REM_CH else 0) for w in range(NWORK)]
_ROW_MAP = np.concatenate(
    [np.minimum(np.arange(s, s + SLOT), NCH - 1) for s in _STARTS])
_PAD_ROW = np.concatenate(
    [np.arange(SLOT) >= c for c in _COUNTS])[:, None]
_PAD_DST = (N + np.arange(CHUNK))[None, :]


TH = T * HID


def _msg0_body(h_ref, wm_ref, bm_ref, out_ref):
    out_ref[...] = lax.dot_general(
        h_ref[...], wm_ref[...], (((1,), (0,)), ((), ())),
        preferred_element_type=jnp.float32) + bm_ref[...]


def _msg0(h, wmcat, bmcat):
    return pl.pallas_call(
        _msg0_body,
        grid=(NB,),
        in_specs=[
            pl.BlockSpec((BLK, HID), lambda i: (i, 0)),
            pl.BlockSpec((HID, TH), lambda i: (0, 0)),
            pl.BlockSpec((1, TH), lambda i: (0, 0)),
        ],
        out_specs=pl.BlockSpec((BLK, TH), lambda i: (i, 0)),
        out_shape=jax.ShapeDtypeStruct((N, TH), jnp.float32),
    )(h, wmcat, bmcat)


def _sc_scatter(hs_flat, gidx_p, dst_p):
    mesh = plsc.VectorSubcoreMesh(core_axis_name="c", subcore_axis_name="s")

    @functools.partial(
        pl.kernel,
        out_type=jax.ShapeDtypeStruct((NCORES, NPAD, HID), jnp.float32),
        mesh=mesh,
        scratch_types=[
            pltpu.VMEM((SLOT, CHUNK), jnp.int32),
            pltpu.VMEM((SLOT, CHUNK), jnp.int32),
            pltpu.VMEM((CHUNK, HID), jnp.float32),
            pltpu.VMEM_SHARED((NPAD, HID), jnp.float32),
        ],
    )
    def k(hs_hbm, gi_hbm, di_hbm, out_hbm, gi_v, di_v, buf, acc):
        c = lax.axis_index("c")
        s = lax.axis_index("s")
        wid = c * NSUB + s

        def zb(kk, _):
            buf[kk // 8, pl.ds((kk % 8) * 16, 16)] = jnp.zeros((16,), jnp.float32)
            return 0
        lax.fori_loop(0, (CHUNK * HID) // 16, zb, 0)
        base = s * ROWS_PER_SUB
        for kk in range(ZCOPY):
            pltpu.sync_copy(buf.at[pl.ds(0, ZROWS)],
                            acc.at[pl.ds(base + kk * ZROWS, ZROWS)])
        plsc.subcore_barrier()

        count = BASE_CH + (wid < REM_CH).astype(jnp.int32)
        pltpu.sync_copy(gi_hbm.at[pl.ds(wid * SLOT, SLOT)], gi_v)
        pltpu.sync_copy(di_hbm.at[pl.ds(wid * SLOT, SLOT)], di_v)

        def body(j, _):
            pltpu.sync_copy(hs_hbm.at[gi_v.at[j]], buf)
            pltpu.sync_copy(buf, acc.at[di_v.at[j]], add=True)
            return 0
        lax.fori_loop(0, count, body, 0)

        plsc.subcore_barrier()
        for kk in range(ZCOPY):
            r0 = base + kk * ZROWS
            pltpu.sync_copy(acc.at[pl.ds(r0, ZROWS)],
                            out_hbm.at[c, pl.ds(r0, ZROWS)])

    return k(hs_flat, gidx_p, dst_p)


def _gru_math(parts_ref, h_ref, wih_ref, whh_ref, bih_ref, bhh_ref):
    a = parts_ref[0] + parts_ref[1]
    h = h_ref[...]
    gi = lax.dot_general(a, wih_ref[...], (((1,), (1,)), ((), ())),
                         preferred_element_type=jnp.float32) + bih_ref[...]
    gh = lax.dot_general(h, whh_ref[...], (((1,), (1,)), ((), ())),
                         preferred_element_type=jnp.float32) + bhh_ref[...]
    r = jax.nn.sigmoid(gi[:, :HID] + gh[:, :HID])
    z = jax.nn.sigmoid(gi[:, HID:2 * HID] + gh[:, HID:2 * HID])
    cc = jnp.tanh(gi[:, 2 * HID:] + r * gh[:, 2 * HID:])
    return (1.0 - z) * cc + z * h


def _step_body(parts_ref, h_ref, wih_ref, whh_ref, bih_ref, bhh_ref,
               wm_ref, bm_ref, hn_ref, hs_ref):
    hn = _gru_math(parts_ref, h_ref, wih_ref, whh_ref, bih_ref, bhh_ref)
    hn_ref[...] = hn
    hs_ref[...] = lax.dot_general(
        hn, wm_ref[...], (((1,), (0,)), ((), ())),
        preferred_element_type=jnp.float32) + bm_ref[...]


def _step(parts, h, wih, whh, bih, bhh, wmcat, bmcat):
    return pl.pallas_call(
        _step_body,
        grid=(NB,),
        in_specs=[
            pl.BlockSpec((NCORES, BLK, HID), lambda i: (0, i, 0)),
            pl.BlockSpec((BLK, HID), lambda i: (i, 0)),
            pl.BlockSpec((3 * HID, HID), lambda i: (0, 0)),
            pl.BlockSpec((3 * HID, HID), lambda i: (0, 0)),
            pl.BlockSpec((1, 3 * HID), lambda i: (0, 0)),
            pl.BlockSpec((1, 3 * HID), lambda i: (0, 0)),
            pl.BlockSpec((HID, TH), lambda i: (0, 0)),
            pl.BlockSpec((1, TH), lambda i: (0, 0)),
        ],
        out_specs=[
            pl.BlockSpec((BLK, HID), lambda i: (i, 0)),
            pl.BlockSpec((BLK, TH), lambda i: (i, 0)),
        ],
        out_shape=[
            jax.ShapeDtypeStruct((N, HID), jnp.float32),
            jax.ShapeDtypeStruct((N, TH), jnp.float32),
        ],
    )(parts, h, wih, whh, bih, bhh, wmcat, bmcat)


def _last_body(parts_ref, h_ref, wih_ref, whh_ref, bih_ref, bhh_ref,
               wcls_ref, bcls_ref, out_ref, acc_ref):
    hn = _gru_math(parts_ref, h_ref, wih_ref, whh_ref, bih_ref, bhh_ref)
    i = pl.program_id(0)

    @pl.when(i == 0)
    def _():
        acc_ref[...] = jnp.zeros_like(acc_ref)

    acc_ref[...] += jnp.sum(jax.nn.relu(hn), axis=0, keepdims=True)

    @pl.when(i == NB - 1)
    def _():
        sv = jnp.sum(acc_ref[...] * wcls_ref[...]) + bcls_ref[0, 0]
        out_ref[...] = jax.nn.sigmoid(sv) * jnp.ones((1, 1), jnp.float32)


def _last(parts, h, wih, whh, bih, bhh, W_cls, b_cls):
    return pl.pallas_call(
        _last_body,
        grid=(NB,),
        in_specs=[
            pl.BlockSpec((NCORES, BLK, HID), lambda i: (0, i, 0)),
            pl.BlockSpec((BLK, HID), lambda i: (i, 0)),
            pl.BlockSpec((3 * HID, HID), lambda i: (0, 0)),
            pl.BlockSpec((3 * HID, HID), lambda i: (0, 0)),
            pl.BlockSpec((1, 3 * HID), lambda i: (0, 0)),
            pl.BlockSpec((1, 3 * HID), lambda i: (0, 0)),
            pl.BlockSpec((1, HID), lambda i: (0, 0)),
            pl.BlockSpec((1, 1), lambda i: (0, 0)),
        ],
        out_specs=pl.BlockSpec((1, 1), lambda i: (0, 0)),
        out_shape=jax.ShapeDtypeStruct((1, 1), jnp.float32),
        scratch_shapes=[pltpu.VMEM((1, HID), jnp.float32)],
    )(parts, h, wih, whh, bih, bhh, W_cls, b_cls.reshape(1, 1))


def kernel(x, edge_index, etypes, W_msg, b_msg, W_ih, W_hh, b_ih, b_hh, W_cls, b_cls):
    src = edge_index[0].astype(jnp.int32)
    dst = edge_index[1].astype(jnp.int32)
    et = etypes.astype(jnp.int32)

    row_map = jnp.asarray(_ROW_MAP, dtype=jnp.int32)
    pad_row = jnp.asarray(_PAD_ROW)
    pad_dst = jnp.asarray(_PAD_DST, dtype=jnp.int32)
    gidx_p = jnp.where(pad_row, 0, (src * T + et).reshape(NCH, CHUNK)[row_map])
    dst_p = jnp.where(pad_row, pad_dst, dst.reshape(NCH, CHUNK)[row_map])

    wmcat = jnp.transpose(W_msg, (2, 0, 1)).reshape(HID, TH)
    bmcat = b_msg.reshape(1, TH)
    bih_r = b_ih.reshape(1, 3 * HID)
    bhh_r = b_hh.reshape(1, 3 * HID)

    h = x
    hs = _msg0(h, wmcat, bmcat)
    for k in range(STEPS):
        parts = _sc_scatter(hs.reshape(N * T, HID), gidx_p, dst_p)
        if k < STEPS - 1:
            h, hs = _step(parts, h, W_ih, W_hh, bih_r, bhh_r, wmcat, bmcat)
        else:
            out = _last(parts, h, W_ih, W_hh, bih_r, bhh_r, W_cls, b_cls)

    return out.reshape(1)


# speedup vs baseline: 2.4794x; 1.1987x over previous
import functools

import jax
import jax.numpy as jnp
import numpy as np
from jax import lax
from jax.experimental import pallas as pl
from jax.experimental.pallas import tpu as pltpu
from jax.experimental.pallas import tpu_sc as plsc

N = 10000
E = 320000
HID = 128
T = 13
STEPS = 6

BLK = 1000
NB = N // BLK
CHUNK = 128
NCH = E // CHUNK
NCORES = 2
NSUB = 16
NWORK = NCORES * NSUB
BASE_CH = NCH // NWORK
REM_CH = NCH % NWORK
SLOT = 80
NPAD = 10240
ROWS_PER_SUB = NPAD // NSUB
ZROWS = 128
ZCOPY = ROWS_PER_SUB // ZROWS

_STARTS = [w * BASE_CH + min(w, REM_CH) for w in range(NWORK)]
_COUNTS = [BASE_CH + (1 if w < REM_CH else 0) for w in range(NWORK)]
_ROW_MAP = np.concatenate(
    [np.minimum(np.arange(s, s + SLOT), NCH - 1) for s in _STARTS])
_PAD_ROW = np.concatenate(
    [np.arange(SLOT) >= c for c in _COUNTS])[:, None]
_PAD_DST = (N + np.arange(CHUNK))[None, :]


TH = T * HID


def _msg0_body(h_ref, wm_ref, bm_ref, out_ref):
    out_ref[...] = lax.dot_general(
        h_ref[...], wm_ref[...], (((1,), (0,)), ((), ())),
        preferred_element_type=jnp.float32) + bm_ref[...]


def _msg0(h, wmcat, bmcat):
    return pl.pallas_call(
        _msg0_body,
        grid=(NB,),
        in_specs=[
            pl.BlockSpec((BLK, HID), lambda i: (i, 0)),
            pl.BlockSpec((HID, TH), lambda i: (0, 0)),
            pl.BlockSpec((1, TH), lambda i: (0, 0)),
        ],
        out_specs=pl.BlockSpec((BLK, TH), lambda i: (i, 0)),
        out_shape=jax.ShapeDtypeStruct((N, TH), jnp.float32),
    )(h, wmcat, bmcat)


def _sc_scatter(hs_flat, gidx_p, dst_p):
    mesh = plsc.VectorSubcoreMesh(core_axis_name="c", subcore_axis_name="s")

    @functools.partial(
        pl.kernel,
        out_type=jax.ShapeDtypeStruct((NCORES, NPAD, HID), jnp.float32),
        mesh=mesh,
        scratch_types=[
            pltpu.VMEM((2, CHUNK), jnp.int32),
            pltpu.VMEM((SLOT, CHUNK), jnp.int32),
            pltpu.VMEM((2, CHUNK, HID), jnp.float32),
            pltpu.VMEM_SHARED((NPAD, HID), jnp.float32),
            pltpu.SemaphoreType.DMA,
            pltpu.SemaphoreType.DMA,
        ],
    )
    def k(hs_hbm, gi_hbm, di_hbm, out_hbm, gi2, di_v, buf2, acc, semI, semG):
        c = lax.axis_index("c")
        s = lax.axis_index("s")
        wid = c * NSUB + s

        def zb(kk, _):
            buf2[0, kk // 8, pl.ds((kk % 8) * 16, 16)] = jnp.zeros((16,), jnp.float32)
            return 0
        lax.fori_loop(0, (CHUNK * HID) // 16, zb, 0)
        base = s * ROWS_PER_SUB
        for kk in range(ZCOPY):
            pltpu.sync_copy(buf2.at[0, pl.ds(0, ZROWS)],
                            acc.at[pl.ds(base + kk * ZROWS, ZROWS)])
        plsc.subcore_barrier()

        count = BASE_CH + (wid < REM_CH).astype(jnp.int32)
        base_c = wid * SLOT
        pltpu.sync_copy(di_hbm.at[pl.ds(base_c, SLOT)], di_v)
        pltpu.sync_copy(gi_hbm.at[pl.ds(base_c * CHUNK, CHUNK)], gi2.at[0])
        pltpu.async_copy(hs_hbm.at[gi2.at[0]], buf2.at[0], semG)
        pltpu.async_copy(gi_hbm.at[pl.ds((base_c + 1) * CHUNK, CHUNK)],
                         gi2.at[1], semI)

        def body(j, _):
            p = jnp.bitwise_and(j, 1)
            pltpu.make_async_copy(gi_hbm.at[pl.ds(0, CHUNK)],
                                  gi2.at[0], semI).wait()
            pltpu.make_async_copy(hs_hbm.at[gi2.at[0]],
                                  buf2.at[0], semG).wait()
            cfetch = base_c + jnp.minimum(j + 2, count - 1)
            pltpu.async_copy(gi_hbm.at[pl.ds(cfetch * CHUNK, CHUNK)],
                             gi2.at[p], semI)
            pltpu.async_copy(hs_hbm.at[gi2.at[1 - p]], buf2.at[1 - p], semG)
            pltpu.sync_copy(buf2.at[p], acc.at[di_v.at[j]], add=True)
            return 0
        lax.fori_loop(0, count, body, 0)
        pltpu.make_async_copy(gi_hbm.at[pl.ds(0, CHUNK)], gi2.at[0], semI).wait()
        pltpu.make_async_copy(hs_hbm.at[gi2.at[0]], buf2.at[0], semG).wait()

        plsc.subcore_barrier()
        for kk in range(ZCOPY):
            r0 = base + kk * ZROWS
            pltpu.sync_copy(acc.at[pl.ds(r0, ZROWS)],
                            out_hbm.at[c, pl.ds(r0, ZROWS)])

    return k(hs_flat, gidx_p, dst_p)


def _gru_math(parts_ref, h_ref, wih_ref, whh_ref, bih_ref, bhh_ref):
    a = parts_ref[0] + parts_ref[1]
    h = h_ref[...]
    gi = lax.dot_general(a, wih_ref[...], (((1,), (1,)), ((), ())),
                         preferred_element_type=jnp.float32) + bih_ref[...]
    gh = lax.dot_general(h, whh_ref[...], (((1,), (1,)), ((), ())),
                         preferred_element_type=jnp.float32) + bhh_ref[...]
    r = jax.nn.sigmoid(gi[:, :HID] + gh[:, :HID])
    z = jax.nn.sigmoid(gi[:, HID:2 * HID] + gh[:, HID:2 * HID])
    cc = jnp.tanh(gi[:, 2 * HID:] + r * gh[:, 2 * HID:])
    return (1.0 - z) * cc + z * h


def _step_body(parts_ref, h_ref, wih_ref, whh_ref, bih_ref, bhh_ref,
               wm_ref, bm_ref, hn_ref, hs_ref):
    hn = _gru_math(parts_ref, h_ref, wih_ref, whh_ref, bih_ref, bhh_ref)
    hn_ref[...] = hn
    hs_ref[...] = lax.dot_general(
        hn, wm_ref[...], (((1,), (0,)), ((), ())),
        preferred_element_type=jnp.float32) + bm_ref[...]


def _step(parts, h, wih, whh, bih, bhh, wmcat, bmcat):
    return pl.pallas_call(
        _step_body,
        grid=(NB,),
        in_specs=[
            pl.BlockSpec((NCORES, BLK, HID), lambda i: (0, i, 0)),
            pl.BlockSpec((BLK, HID), lambda i: (i, 0)),
            pl.BlockSpec((3 * HID, HID), lambda i: (0, 0)),
            pl.BlockSpec((3 * HID, HID), lambda i: (0, 0)),
            pl.BlockSpec((1, 3 * HID), lambda i: (0, 0)),
            pl.BlockSpec((1, 3 * HID), lambda i: (0, 0)),
            pl.BlockSpec((HID, TH), lambda i: (0, 0)),
            pl.BlockSpec((1, TH), lambda i: (0, 0)),
        ],
        out_specs=[
            pl.BlockSpec((BLK, HID), lambda i: (i, 0)),
            pl.BlockSpec((BLK, TH), lambda i: (i, 0)),
        ],
        out_shape=[
            jax.ShapeDtypeStruct((N, HID), jnp.float32),
            jax.ShapeDtypeStruct((N, TH), jnp.float32),
        ],
    )(parts, h, wih, whh, bih, bhh, wmcat, bmcat)


def _last_body(parts_ref, h_ref, wih_ref, whh_ref, bih_ref, bhh_ref,
               wcls_ref, bcls_ref, out_ref, acc_ref):
    hn = _gru_math(parts_ref, h_ref, wih_ref, whh_ref, bih_ref, bhh_ref)
    i = pl.program_id(0)

    @pl.when(i == 0)
    def _():
        acc_ref[...] = jnp.zeros_like(acc_ref)

    acc_ref[...] += jnp.sum(jax.nn.relu(hn), axis=0, keepdims=True)

    @pl.when(i == NB - 1)
    def _():
        sv = jnp.sum(acc_ref[...] * wcls_ref[...]) + bcls_ref[0, 0]
        out_ref[...] = jax.nn.sigmoid(sv) * jnp.ones((1, 1), jnp.float32)


def _last(parts, h, wih, whh, bih, bhh, W_cls, b_cls):
    return pl.pallas_call(
        _last_body,
        grid=(NB,),
        in_specs=[
            pl.BlockSpec((NCORES, BLK, HID), lambda i: (0, i, 0)),
            pl.BlockSpec((BLK, HID), lambda i: (i, 0)),
            pl.BlockSpec((3 * HID, HID), lambda i: (0, 0)),
            pl.BlockSpec((3 * HID, HID), lambda i: (0, 0)),
            pl.BlockSpec((1, 3 * HID), lambda i: (0, 0)),
            pl.BlockSpec((1, 3 * HID), lambda i: (0, 0)),
            pl.BlockSpec((1, HID), lambda i: (0, 0)),
            pl.BlockSpec((1, 1), lambda i: (0, 0)),
        ],
        out_specs=pl.BlockSpec((1, 1), lambda i: (0, 0)),
        out_shape=jax.ShapeDtypeStruct((1, 1), jnp.float32),
        scratch_shapes=[pltpu.VMEM((1, HID), jnp.float32)],
    )(parts, h, wih, whh, bih, bhh, W_cls, b_cls.reshape(1, 1))


def kernel(x, edge_index, etypes, W_msg, b_msg, W_ih, W_hh, b_ih, b_hh, W_cls, b_cls):
    src = edge_index[0].astype(jnp.int32)
    dst = edge_index[1].astype(jnp.int32)
    et = etypes.astype(jnp.int32)

    row_map = jnp.asarray(_ROW_MAP, dtype=jnp.int32)
    pad_row = jnp.asarray(_PAD_ROW)
    pad_dst = jnp.asarray(_PAD_DST, dtype=jnp.int32)
    gidx_p = jnp.where(pad_row, 0,
                       (src * T + et).reshape(NCH, CHUNK)[row_map]).reshape(-1)
    dst_p = jnp.where(pad_row, pad_dst, dst.reshape(NCH, CHUNK)[row_map])

    wmcat = jnp.transpose(W_msg, (2, 0, 1)).reshape(HID, TH)
    bmcat = b_msg.reshape(1, TH)
    bih_r = b_ih.reshape(1, 3 * HID)
    bhh_r = b_hh.reshape(1, 3 * HID)

    h = x
    hs = _msg0(h, wmcat, bmcat)
    for k in range(STEPS):
        parts = _sc_scatter(hs.reshape(N * T, HID), gidx_p, dst_p)
        if k < STEPS - 1:
            h, hs = _step(parts, h, W_ih, W_hh, bih_r, bhh_r, wmcat, bmcat)
        else:
            out = _last(parts, h, W_ih, W_hh, bih_r, bhh_r, W_cls, b_cls)

    return out.reshape(1)


# speedup vs baseline: 2.4849x; 1.0022x over previous
import functools

import jax
import jax.numpy as jnp
import numpy as np
from jax import lax
from jax.experimental import pallas as pl
from jax.experimental.pallas import tpu as pltpu
from jax.experimental.pallas import tpu_sc as plsc

N = 10000
E = 320000
HID = 128
T = 13
STEPS = 6

BLK = 1000
NB = N // BLK
CHUNK = 128
NCH = E // CHUNK
NCORES = 2
NSUB = 16
NWORK = NCORES * NSUB
BASE_CH = NCH // NWORK
REM_CH = NCH % NWORK
SLOT = 80
NPAD = 10240
ROWS_PER_SUB = NPAD // NSUB
ZROWS = 128
ZCOPY = ROWS_PER_SUB // ZROWS

_STARTS = [w * BASE_CH + min(w, REM_CH) for w in range(NWORK)]
_COUNTS = [BASE_CH + (1 if w < REM_CH else 0) for w in range(NWORK)]
_ROW_MAP = np.concatenate(
    [np.minimum(np.arange(s, s + SLOT), NCH - 1) for s in _STARTS])
_PAD_ROW = np.concatenate(
    [np.arange(SLOT) >= c for c in _COUNTS])[:, None]
_PAD_DST = (N + np.arange(CHUNK))[None, :]


TH = T * HID


def _msg0_body(h_ref, wm_ref, bm_ref, out_ref):
    out_ref[...] = lax.dot_general(
        h_ref[...].astype(jnp.bfloat16), wm_ref[...], (((1,), (0,)), ((), ())),
        preferred_element_type=jnp.float32) + bm_ref[...]


def _msg0(h, wmcat, bmcat):
    return pl.pallas_call(
        _msg0_body,
        grid=(NB,),
        in_specs=[
            pl.BlockSpec((BLK, HID), lambda i: (i, 0)),
            pl.BlockSpec((HID, TH), lambda i: (0, 0)),
            pl.BlockSpec((1, TH), lambda i: (0, 0)),
        ],
        out_specs=pl.BlockSpec((BLK, TH), lambda i: (i, 0)),
        out_shape=jax.ShapeDtypeStruct((N, TH), jnp.float32),
    )(h, wmcat, bmcat)


def _sc_scatter(hs_flat, gidx_p, dst_p):
    mesh = plsc.VectorSubcoreMesh(core_axis_name="c", subcore_axis_name="s")

    @functools.partial(
        pl.kernel,
        out_type=jax.ShapeDtypeStruct((NCORES, NPAD, HID), jnp.float32),
        mesh=mesh,
        scratch_types=[
            pltpu.VMEM((2, CHUNK), jnp.int32),
            pltpu.VMEM((SLOT, CHUNK), jnp.int32),
            pltpu.VMEM((2, CHUNK, HID), jnp.float32),
            pltpu.VMEM_SHARED((NPAD, HID), jnp.float32),
            pltpu.SemaphoreType.DMA,
            pltpu.SemaphoreType.DMA,
        ],
    )
    def k(hs_hbm, gi_hbm, di_hbm, out_hbm, gi2, di_v, buf2, acc, semI, semG):
        c = lax.axis_index("c")
        s = lax.axis_index("s")
        wid = c * NSUB + s

        def zb(kk, _):
            buf2[0, kk // 8, pl.ds((kk % 8) * 16, 16)] = jnp.zeros((16,), jnp.float32)
            return 0
        lax.fori_loop(0, (CHUNK * HID) // 16, zb, 0)
        base = s * ROWS_PER_SUB
        for kk in range(ZCOPY):
            pltpu.sync_copy(buf2.at[0, pl.ds(0, ZROWS)],
                            acc.at[pl.ds(base + kk * ZROWS, ZROWS)])
        plsc.subcore_barrier()

        count = BASE_CH + (wid < REM_CH).astype(jnp.int32)
        base_c = wid * SLOT
        pltpu.sync_copy(di_hbm.at[pl.ds(base_c, SLOT)], di_v)
        pltpu.sync_copy(gi_hbm.at[pl.ds(base_c * CHUNK, CHUNK)], gi2.at[0])
        pltpu.async_copy(hs_hbm.at[gi2.at[0]], buf2.at[0], semG)
        pltpu.async_copy(gi_hbm.at[pl.ds((base_c + 1) * CHUNK, CHUNK)],
                         gi2.at[1], semI)

        def body(j, _):
            p = jnp.bitwise_and(j, 1)
            pltpu.make_async_copy(gi_hbm.at[pl.ds(0, CHUNK)],
                                  gi2.at[0], semI).wait()
            pltpu.make_async_copy(hs_hbm.at[gi2.at[0]],
                                  buf2.at[0], semG).wait()
            cfetch = base_c + jnp.minimum(j + 2, count - 1)
            pltpu.async_copy(gi_hbm.at[pl.ds(cfetch * CHUNK, CHUNK)],
                             gi2.at[p], semI)
            pltpu.async_copy(hs_hbm.at[gi2.at[1 - p]], buf2.at[1 - p], semG)
            pltpu.sync_copy(buf2.at[p], acc.at[di_v.at[j]], add=True)
            return 0
        lax.fori_loop(0, count, body, 0)
        pltpu.make_async_copy(gi_hbm.at[pl.ds(0, CHUNK)], gi2.at[0], semI).wait()
        pltpu.make_async_copy(hs_hbm.at[gi2.at[0]], buf2.at[0], semG).wait()

        plsc.subcore_barrier()
        for kk in range(ZCOPY):
            r0 = base + kk * ZROWS
            pltpu.sync_copy(acc.at[pl.ds(r0, ZROWS)],
                            out_hbm.at[c, pl.ds(r0, ZROWS)])

    return k(hs_flat, gidx_p, dst_p)


def _gru_math(parts_ref, h_ref, wih_ref, whh_ref, bih_ref, bhh_ref):
    a = parts_ref[0] + parts_ref[1]
    h = h_ref[...]
    gi = lax.dot_general(a.astype(jnp.bfloat16), wih_ref[...],
                         (((1,), (1,)), ((), ())),
                         preferred_element_type=jnp.float32) + bih_ref[...]
    gh = lax.dot_general(h.astype(jnp.bfloat16), whh_ref[...],
                         (((1,), (1,)), ((), ())),
                         preferred_element_type=jnp.float32) + bhh_ref[...]
    r = jax.nn.sigmoid(gi[:, :HID] + gh[:, :HID])
    z = jax.nn.sigmoid(gi[:, HID:2 * HID] + gh[:, HID:2 * HID])
    cc = jnp.tanh(gi[:, 2 * HID:] + r * gh[:, 2 * HID:])
    return (1.0 - z) * cc + z * h


def _step_body(parts_ref, h_ref, wih_ref, whh_ref, bih_ref, bhh_ref,
               wm_ref, bm_ref, hn_ref, hs_ref):
    hn = _gru_math(parts_ref, h_ref, wih_ref, whh_ref, bih_ref, bhh_ref)
    hn_ref[...] = hn
    hs_ref[...] = lax.dot_general(
        hn.astype(jnp.bfloat16), wm_ref[...], (((1,), (0,)), ((), ())),
        preferred_element_type=jnp.float32) + bm_ref[...]


def _step(parts, h, wih, whh, bih, bhh, wmcat, bmcat):
    return pl.pallas_call(
        _step_body,
        grid=(NB,),
        in_specs=[
            pl.BlockSpec((NCORES, BLK, HID), lambda i: (0, i, 0)),
            pl.BlockSpec((BLK, HID), lambda i: (i, 0)),
            pl.BlockSpec((3 * HID, HID), lambda i: (0, 0)),
            pl.BlockSpec((3 * HID, HID), lambda i: (0, 0)),
            pl.BlockSpec((1, 3 * HID), lambda i: (0, 0)),
            pl.BlockSpec((1, 3 * HID), lambda i: (0, 0)),
            pl.BlockSpec((HID, TH), lambda i: (0, 0)),
            pl.BlockSpec((1, TH), lambda i: (0, 0)),
        ],
        out_specs=[
            pl.BlockSpec((BLK, HID), lambda i: (i, 0)),
            pl.BlockSpec((BLK, TH), lambda i: (i, 0)),
        ],
        out_shape=[
            jax.ShapeDtypeStruct((N, HID), jnp.float32),
            jax.ShapeDtypeStruct((N, TH), jnp.float32),
        ],
    )(parts, h, wih, whh, bih, bhh, wmcat, bmcat)


def _last_body(parts_ref, h_ref, wih_ref, whh_ref, bih_ref, bhh_ref,
               wcls_ref, bcls_ref, out_ref, acc_ref):
    hn = _gru_math(parts_ref, h_ref, wih_ref, whh_ref, bih_ref, bhh_ref)
    i = pl.program_id(0)

    @pl.when(i == 0)
    def _():
        acc_ref[...] = jnp.zeros_like(acc_ref)

    acc_ref[...] += jnp.sum(jax.nn.relu(hn), axis=0, keepdims=True)

    @pl.when(i == NB - 1)
    def _():
        sv = jnp.sum(acc_ref[...] * wcls_ref[...]) + bcls_ref[0, 0]
        out_ref[...] = jax.nn.sigmoid(sv) * jnp.ones((1, 1), jnp.float32)


def _last(parts, h, wih, whh, bih, bhh, W_cls, b_cls):
    return pl.pallas_call(
        _last_body,
        grid=(NB,),
        in_specs=[
            pl.BlockSpec((NCORES, BLK, HID), lambda i: (0, i, 0)),
            pl.BlockSpec((BLK, HID), lambda i: (i, 0)),
            pl.BlockSpec((3 * HID, HID), lambda i: (0, 0)),
            pl.BlockSpec((3 * HID, HID), lambda i: (0, 0)),
            pl.BlockSpec((1, 3 * HID), lambda i: (0, 0)),
            pl.BlockSpec((1, 3 * HID), lambda i: (0, 0)),
            pl.BlockSpec((1, HID), lambda i: (0, 0)),
            pl.BlockSpec((1, 1), lambda i: (0, 0)),
        ],
        out_specs=pl.BlockSpec((1, 1), lambda i: (0, 0)),
        out_shape=jax.ShapeDtypeStruct((1, 1), jnp.float32),
        scratch_shapes=[pltpu.VMEM((1, HID), jnp.float32)],
    )(parts, h, wih, whh, bih, bhh, W_cls, b_cls.reshape(1, 1))


def kernel(x, edge_index, etypes, W_msg, b_msg, W_ih, W_hh, b_ih, b_hh, W_cls, b_cls):
    src = edge_index[0].astype(jnp.int32)
    dst = edge_index[1].astype(jnp.int32)
    et = etypes.astype(jnp.int32)

    row_map = jnp.asarray(_ROW_MAP, dtype=jnp.int32)
    pad_row = jnp.asarray(_PAD_ROW)
    pad_dst = jnp.asarray(_PAD_DST, dtype=jnp.int32)
    gidx_p = jnp.where(pad_row, 0,
                       (src * T + et).reshape(NCH, CHUNK)[row_map]).reshape(-1)
    dst_p = jnp.where(pad_row, pad_dst, dst.reshape(NCH, CHUNK)[row_map])

    wmcat = jnp.transpose(W_msg, (2, 0, 1)).reshape(HID, TH).astype(jnp.bfloat16)
    bmcat = b_msg.reshape(1, TH)
    wih_b = W_ih.astype(jnp.bfloat16)
    whh_b = W_hh.astype(jnp.bfloat16)
    bih_r = b_ih.reshape(1, 3 * HID)
    bhh_r = b_hh.reshape(1, 3 * HID)

    h = x
    hs = _msg0(h, wmcat, bmcat)
    for k in range(STEPS):
        parts = _sc_scatter(hs.reshape(N * T, HID), gidx_p, dst_p)
        if k < STEPS - 1:
            h, hs = _step(parts, h, wih_b, whh_b, bih_r, bhh_r, wmcat, bmcat)
        else:
            out = _last(parts, h, wih_b, whh_b, bih_r, bhh_r, W_cls, b_cls)

    return out.reshape(1)
